# baseline (speedup 1.0000x reference)
import functools

import jax
import jax.numpy as jnp
from jax import lax
from jax.experimental import pallas as pl
from jax.experimental.pallas import tpu as pltpu
from jax.experimental.pallas import tpu_sc as plsc

B, L, V = 32, 16, 100000
MASK_TOKEN_ID = V - 1
DYNAMIC_THRESHOLD = 0.9
ROWS = B * L
R = 16
NSTEP = ROWS // R


def _stage1_body(temp_ref, logits_ref, gumb_ref, x0_ref, s_ref, sat_ref):
    t = temp_ref[0, 0, :]
    lg = logits_ref[...]
    gu = gumb_ref[...]
    scaled = lg / t[:, None]
    z = scaled - jnp.log(-jnp.log(gu))
    idx = jnp.argmax(z, axis=1).astype(jnp.int32)
    s = jnp.sum(jnp.exp(scaled), axis=1)
    col = jax.lax.broadcasted_iota(jnp.int32, (R, V), 1)
    scaled_at = jnp.sum(jnp.where(col == idx[:, None], scaled, 0.0), axis=1)
    x0_ref[0, 0, :] = idx
    s_ref[0, 0, :] = s
    sat_ref[0, 0, :] = scaled_at


_GATHER_DNUMS = lax.GatherDimensionNumbers(
    offset_dims=(), collapsed_slice_dims=(0,), start_index_map=(0,))


def _lane_perm(v, idx):
    return lax.gather(v, idx[:, None], _GATHER_DNUMS, slice_sizes=(1,),
                      mode=lax.GatherScatterMode.PROMISE_IN_BOUNDS)


def _allmax(v):
    io = lax.iota(jnp.int32, L)
    for d in (8, 4, 2, 1):
        v = jnp.maximum(v, _lane_perm(v, io ^ d))
    return v


def _cumsum_lanes(v):
    io = lax.iota(jnp.int32, L)
    for d in (1, 2, 4, 8):
        shifted = _lane_perm(v, (io - d) & (L - 1))
        v = v + jnp.where(io >= d, shifted, 0)
    return v


def _sc_transfer_body(x_hbm, x0_hbm, sat_hbm, s_hbm,
                      xnew_hbm, ti_hbm, p_hbm,
                      x_v, x0_v, sat_v, s_v, o_v):
    wid = lax.axis_index("s") * 2 + lax.axis_index("c")
    base = wid * L
    pltpu.sync_copy(x_hbm.at[pl.ds(base, L)], x_v)
    pltpu.sync_copy(x0_hbm.at[pl.ds(base, L)], x0_v)
    pltpu.sync_copy(sat_hbm.at[pl.ds(base, L)], sat_v)
    pltpu.sync_copy(s_hbm.at[pl.ds(base, L)], s_v)
    p = jnp.exp(sat_v[...]) / s_v[...]
    x = x_v[...]
    is_mask = x == MASK_TOKEN_ID
    conf = jnp.where(is_mask, p, -jnp.inf)
    high_i = jnp.where(conf > DYNAMIC_THRESHOLD, 1, 0)
    mask_i = jnp.where(is_mask, 1, 0)
    hh = _allmax(high_i)
    am = _allmax(mask_i)
    cmax = _allmax(conf)
    hit_i = jnp.where(conf == cmax, 1, 0)
    top1_i = hit_i * jnp.where(_cumsum_lanes(hit_i) == 1, 1, 0)
    ti_i = (high_i * hh + top1_i * (1 - hh)) * am
    sat_v[...] = p
    pltpu.sync_copy(sat_v, p_hbm.at[pl.ds(base, L)])
    o_v[...] = x0_v[...] * ti_i + x * (1 - ti_i)
    pltpu.sync_copy(o_v, xnew_hbm.at[pl.ds(base, L)])
    o_v[...] = ti_i
    pltpu.sync_copy(o_v, ti_hbm.at[pl.ds(base, L)])


def _count_body(ti_ref, num_ref):
    num_ref[...] = jnp.sum(ti_ref[...], keepdims=True).reshape(1, 1)


@functools.partial(jax.jit, static_argnames=("interpret",))
def kernel(logits, temperatures, gumbel_u, x, interpret=False):
    lg = logits.reshape(ROWS, V)
    gu = gumbel_u.reshape(ROWS, V)
    trow = jnp.repeat(temperatures, L)

    x0r, sr, satr = pl.pallas_call(
        _stage1_body,
        grid=(NSTEP,),
        in_specs=[
            pl.BlockSpec((1, 1, R), lambda i: (i, 0, 0)),
            pl.BlockSpec((R, V), lambda i: (i, 0)),
            pl.BlockSpec((R, V), lambda i: (i, 0)),
        ],
        out_specs=[
            pl.BlockSpec((1, 1, R), lambda i: (i, 0, 0)),
            pl.BlockSpec((1, 1, R), lambda i: (i, 0, 0)),
            pl.BlockSpec((1, 1, R), lambda i: (i, 0, 0)),
        ],
        out_shape=[
            jax.ShapeDtypeStruct((NSTEP, 1, R), jnp.int32),
            jax.ShapeDtypeStruct((NSTEP, 1, R), jnp.float32),
            jax.ShapeDtypeStruct((NSTEP, 1, R), jnp.float32),
        ],
        interpret=interpret,
    )(trow.reshape(NSTEP, 1, R), lg, gu)

    x0 = x0r.reshape(B, L)

    sc_transfer = pl.kernel(
        _sc_transfer_body,
        out_type=[
            jax.ShapeDtypeStruct((ROWS,), jnp.int32),
            jax.ShapeDtypeStruct((ROWS,), jnp.int32),
            jax.ShapeDtypeStruct((ROWS,), jnp.float32),
        ],
        mesh=plsc.VectorSubcoreMesh(core_axis_name="c", subcore_axis_name="s"),
        scratch_types=[
            pltpu.VMEM((L,), jnp.int32),
            pltpu.VMEM((L,), jnp.int32),
            pltpu.VMEM((L,), jnp.float32),
            pltpu.VMEM((L,), jnp.float32),
            pltpu.VMEM((L,), jnp.int32),
        ],
    )
    xnew_flat, ti_flat, p_flat = sc_transfer(
        x.reshape(ROWS), x0r.reshape(ROWS), satr.reshape(ROWS),
        sr.reshape(ROWS))

    ti = ti_flat.reshape(B, L)
    num = pl.pallas_call(
        _count_body,
        out_shape=jax.ShapeDtypeStruct((1, 1), jnp.int32),
        interpret=interpret,
    )(ti)

    return (num.reshape(()), xnew_flat.reshape(B, L), x0,
            p_flat.reshape(B, L), ti.astype(jnp.bool_))


# speedup vs baseline: 1.0195x; 1.0195x over previous
import functools

import jax
import jax.numpy as jnp
from jax import lax
from jax.experimental import pallas as pl
from jax.experimental.pallas import tpu as pltpu
from jax.experimental.pallas import tpu_sc as plsc

B, L, V = 32, 16, 100000
MASK_TOKEN_ID = V - 1
DYNAMIC_THRESHOLD = 0.9
ROWS = B * L
R = 16
NSTEP = ROWS // R


def _stage1_body(temp_ref, logits_ref, gumb_ref, pack_ref):
    t = temp_ref[0, 0, :]
    lg = logits_ref[...]
    gu = gumb_ref[...]
    scaled = lg / t[:, None]
    z = scaled - jnp.log(-jnp.log(gu))
    idx = jnp.argmax(z, axis=1).astype(jnp.int32)
    s = jnp.sum(jnp.exp(scaled), axis=1)
    col = jax.lax.broadcasted_iota(jnp.int32, (R, V), 1)
    scaled_at = jnp.sum(jnp.where(col == idx[:, None], scaled, 0.0), axis=1)
    pack_ref[0, 0, :] = idx.astype(jnp.float32)
    pack_ref[0, 1, :] = scaled_at
    pack_ref[0, 2, :] = s


_GATHER_DNUMS = lax.GatherDimensionNumbers(
    offset_dims=(), collapsed_slice_dims=(0,), start_index_map=(0,))


def _lane_perm(v, idx):
    return lax.gather(v, idx[:, None], _GATHER_DNUMS, slice_sizes=(1,),
                      mode=lax.GatherScatterMode.PROMISE_IN_BOUNDS)


def _allmax(v):
    io = lax.iota(jnp.int32, L)
    for d in (8, 4, 2, 1):
        v = jnp.maximum(v, _lane_perm(v, io ^ d))
    return v


def _cumsum_lanes(v):
    io = lax.iota(jnp.int32, L)
    for d in (1, 2, 4, 8):
        shifted = _lane_perm(v, (io - d) & (L - 1))
        v = v + jnp.where(io >= d, shifted, 0)
    return v


def _sc_transfer_body(x_hbm, pack_hbm,
                      xnew_hbm, ti_hbm, p_hbm,
                      x_v, x0_v, sat_v, s_v, o_v):
    wid = lax.axis_index("s") * 2 + lax.axis_index("c")
    base = wid * L
    pltpu.sync_copy(x_hbm.at[pl.ds(base, L)], x_v)
    pltpu.sync_copy(pack_hbm.at[pl.ds(3 * base, L)], x0_v)
    pltpu.sync_copy(pack_hbm.at[pl.ds(3 * base + L, L)], sat_v)
    pltpu.sync_copy(pack_hbm.at[pl.ds(3 * base + 2 * L, L)], s_v)
    p = jnp.exp(sat_v[...]) / s_v[...]
    x = x_v[...]
    is_mask = x == MASK_TOKEN_ID
    conf = jnp.where(is_mask, p, -jnp.inf)
    high_i = jnp.where(conf > DYNAMIC_THRESHOLD, 1, 0)
    mask_i = jnp.where(is_mask, 1, 0)
    hh = _allmax(high_i)
    am = _allmax(mask_i)
    cmax = _allmax(conf)
    hit_i = jnp.where(conf == cmax, 1, 0)
    top1_i = hit_i * jnp.where(_cumsum_lanes(hit_i) == 1, 1, 0)
    ti_i = (high_i * hh + top1_i * (1 - hh)) * am
    sat_v[...] = p
    pltpu.sync_copy(sat_v, p_hbm.at[pl.ds(base, L)])
    x0 = x0_v[...].astype(jnp.int32)
    o_v[...] = x0 * ti_i + x * (1 - ti_i)
    pltpu.sync_copy(o_v, xnew_hbm.at[pl.ds(base, L)])
    o_v[...] = ti_i
    pltpu.sync_copy(o_v, ti_hbm.at[pl.ds(base, L)])


def _count_body(ti_ref, num_ref):
    num_ref[...] = jnp.sum(ti_ref[...], keepdims=True).reshape(1, 1)


@functools.partial(jax.jit, static_argnames=("interpret",))
def kernel(logits, temperatures, gumbel_u, x, interpret=False):
    lg = logits.reshape(ROWS, V)
    gu = gumbel_u.reshape(ROWS, V)
    trow = jnp.repeat(temperatures, L)

    packed = pl.pallas_call(
        _stage1_body,
        grid=(NSTEP,),
        in_specs=[
            pl.BlockSpec((1, 1, R), lambda i: (i, 0, 0)),
            pl.BlockSpec((R, V), lambda i: (i, 0)),
            pl.BlockSpec((R, V), lambda i: (i, 0)),
        ],
        out_specs=[
            pl.BlockSpec((1, 3, R), lambda i: (i, 0, 0)),
        ],
        out_shape=[
            jax.ShapeDtypeStruct((NSTEP, 3, R), jnp.float32),
        ],
        interpret=interpret,
    )(trow.reshape(NSTEP, 1, R), lg, gu)[0]

    x0 = packed[:, 0, :].astype(jnp.int32).reshape(B, L)

    sc_transfer = pl.kernel(
        _sc_transfer_body,
        out_type=[
            jax.ShapeDtypeStruct((ROWS,), jnp.int32),
            jax.ShapeDtypeStruct((ROWS,), jnp.int32),
            jax.ShapeDtypeStruct((ROWS,), jnp.float32),
        ],
        mesh=plsc.VectorSubcoreMesh(core_axis_name="c", subcore_axis_name="s"),
        scratch_types=[
            pltpu.VMEM((L,), jnp.int32),
            pltpu.VMEM((L,), jnp.float32),
            pltpu.VMEM((L,), jnp.float32),
            pltpu.VMEM((L,), jnp.float32),
            pltpu.VMEM((L,), jnp.int32),
        ],
    )
    xnew_flat, ti_flat, p_flat = sc_transfer(
        x.reshape(ROWS), packed.reshape(NSTEP * 3 * R))

    ti = ti_flat.reshape(B, L)
    num = pl.pallas_call(
        _count_body,
        out_shape=jax.ShapeDtypeStruct((1, 1), jnp.int32),
        interpret=interpret,
    )(ti)

    return (num.reshape(()), xnew_flat.reshape(B, L), x0,
            p_flat.reshape(B, L), ti.astype(jnp.bool_))
